# speedup vs baseline: 7.0819x; 7.0819x over previous
import jax
import jax.numpy as jnp
from jax.experimental import pallas as pl
from jax.experimental.pallas import tpu as pltpu

_SMOOTHING = 0.1
_PAD = 0
_V = 32000
_ON = 1.0 - _SMOOTHING
_BASE = _SMOOTHING / (_V - 2)

_BB = 256
_VB = 3200


def _fill_body(tgt_ref, out_ref):
    j = pl.program_id(1)
    tgt = tgt_ref[...]
    col = jax.lax.broadcasted_iota(jnp.int32, (_BB, _VB), 1) + j * _VB
    val = jnp.where(col == tgt, _ON, _BASE)
    dead = (tgt == _PAD) | (col == _PAD)
    out_ref[...] = jnp.where(dead, 0.0, val)


def kernel(tgt_ids):
    B = tgt_ids.shape[0]
    grid = (B // _BB, _V // _VB)
    return pl.pallas_call(
        _fill_body,
        grid=grid,
        in_specs=[pl.BlockSpec((_BB, 1), lambda i, j: (i, 0))],
        out_specs=pl.BlockSpec((_BB, _VB), lambda i, j: (i, j)),
        out_shape=jax.ShapeDtypeStruct((B, _V), jnp.float32),
        compiler_params=pltpu.CompilerParams(
            dimension_semantics=("parallel", "parallel"),
        ),
    )(tgt_ids)
